# baseline (speedup 1.0000x reference)
import functools

import jax
import jax.numpy as jnp
from jax import lax
from jax.experimental import pallas as pl
from jax.experimental.pallas import tpu as pltpu
from jax.experimental.pallas import tpu_sc as plsc

_CHUNK = 128
_NC = 2
_NS = 16


@functools.lru_cache(maxsize=None)
def _make_gather(V, D, B):
    NW = _NC * _NS
    assert B % (NW * _CHUNK) == 0, (B, NW * _CHUNK)
    n_chunk = B // (NW * _CHUNK)
    b_per_w = n_chunk * _CHUNK
    mesh = plsc.VectorSubcoreMesh(core_axis_name="c", subcore_axis_name="s")

    @functools.partial(
        pl.kernel,
        mesh=mesh,
        out_type=jax.ShapeDtypeStruct((B, D), jnp.float32),
        scratch_types=[
            pltpu.VMEM((n_chunk, _CHUNK), jnp.int32),
            pltpu.VMEM((_CHUNK, D), jnp.float32),
            pltpu.SemaphoreType.DMA,
        ],
        compiler_params=pltpu.CompilerParams(use_tc_tiling_on_sc=False),
    )
    def gather_kernel(idx_hbm, table_hbm, out_hbm, idx_v, buf, sem):
        wid = lax.axis_index("s") * _NC + lax.axis_index("c")
        pltpu.sync_copy(idx_hbm.at[wid], idx_v)
        base = wid * b_per_w

        def body(j, carry):
            pltpu.async_copy(table_hbm.at[idx_v.at[j]], buf, sem).wait()
            pltpu.sync_copy(buf, out_hbm.at[pl.ds(base + j * _CHUNK, _CHUNK)])
            return carry

        lax.fori_loop(0, n_chunk, body, 0)

    return gather_kernel


def kernel(x, weight):
    Bt, F = x.shape
    V, D = weight.shape
    B = Bt * F
    NW = _NC * _NS
    idx = x.astype(jnp.int32).reshape(NW, B // (NW * _CHUNK), _CHUNK)
    out = _make_gather(V, D, B)(idx, weight)
    return out.reshape(Bt, F, D)


# speedup vs baseline: 1.0228x; 1.0228x over previous
import functools

import jax
import jax.numpy as jnp
from jax import lax
from jax.experimental import pallas as pl
from jax.experimental.pallas import tpu as pltpu
from jax.experimental.pallas import tpu_sc as plsc

_CHUNK = 128
_NC = 2
_NS = 16


@functools.lru_cache(maxsize=None)
def _make_gather(V, D, B):
    NW = _NC * _NS
    assert B % (NW * _CHUNK) == 0, (B, NW * _CHUNK)
    n_chunk = B // (NW * _CHUNK)
    b_per_w = n_chunk * _CHUNK
    mesh = plsc.VectorSubcoreMesh(core_axis_name="c", subcore_axis_name="s")

    nbuf = min(13, n_chunk)

    @functools.partial(
        pl.kernel,
        mesh=mesh,
        out_type=jax.ShapeDtypeStruct((B, D), jnp.float32),
        scratch_types=(
            [pltpu.VMEM((n_chunk, _CHUNK), jnp.int32)]
            + [pltpu.VMEM((_CHUNK, D), jnp.float32) for _ in range(nbuf)]
            + [pltpu.SemaphoreType.DMA for _ in range(nbuf)]
        ),
        compiler_params=pltpu.CompilerParams(use_tc_tiling_on_sc=False),
    )
    def gather_kernel(idx_hbm, table_hbm, out_hbm, idx_v, *rest):
        bufs, sems = rest[:nbuf], rest[nbuf:]
        wid = lax.axis_index("s") * _NC + lax.axis_index("c")
        pltpu.sync_copy(idx_hbm.at[wid], idx_v)
        base = wid * b_per_w

        handles = [
            pltpu.async_copy(table_hbm.at[idx_v.at[j]], bufs[j], sems[j])
            for j in range(nbuf)
        ]
        for j in range(n_chunk):
            b = j % nbuf
            handles[b].wait()
            pltpu.sync_copy(bufs[b], out_hbm.at[pl.ds(base + j * _CHUNK, _CHUNK)])
            nj = j + nbuf
            if nj < n_chunk:
                handles[b] = pltpu.async_copy(
                    table_hbm.at[idx_v.at[nj]], bufs[b], sems[b]
                )

    return gather_kernel


def kernel(x, weight):
    Bt, F = x.shape
    V, D = weight.shape
    B = Bt * F
    NW = _NC * _NS
    idx = x.astype(jnp.int32).reshape(NW, B // (NW * _CHUNK), _CHUNK)
    out = _make_gather(V, D, B)(idx, weight)
    return out.reshape(Bt, F, D)
